# baseline (speedup 1.0000x reference)
import functools

import jax
import jax.numpy as jnp
from jax import lax
from jax.experimental import pallas as pl
from jax.experimental.pallas import tpu as pltpu
from jax.experimental.pallas import tpu_sc as plsc

N = 16384
VOCAB = 100000
VMAIN = (VOCAB // 128) * 128
VTAIL = VOCAB - VMAIN
EMB_DIM = 64
CHG = 8
OUT_DIM = EMB_DIM + CHG
NC, NS = 2, 16
NW = NC * NS
L = 16
IDX_CHUNK = 2048
NCB = N // IDX_CHUNK
ROUNDS = EMB_DIM // NW
CHG_COLS = N // NW // 2


@functools.partial(
    pl.kernel,
    mesh=plsc.VectorSubcoreMesh(core_axis_name="c", subcore_axis_name="s"),
    out_type=jax.ShapeDtypeStruct((OUT_DIM, N), jnp.float32),
    scratch_types=[
        pltpu.VMEM((ROUNDS, 1), jnp.int32),
        pltpu.VMEM((2, IDX_CHUNK), jnp.int32),
        pltpu.VMEM((1, N), jnp.float32),
        pltpu.VMEM((CHG, CHG_COLS), jnp.float32),
        pltpu.VMEM((ROUNDS, VTAIL), jnp.float32),
        pltpu.VMEM((1, VOCAB), jnp.float32),
        pltpu.VMEM_SHARED((N,), jnp.int32),
        pltpu.SemaphoreType.DMA,
        pltpu.SemaphoreType.DMA,
        pltpu.SemaphoreType.DMA,
        pltpu.SemaphoreType.DMA,
    ],
    compiler_params=pltpu.CompilerParams(
        needs_layout_passes=False,
        disable_bounds_checks=True,
        disable_semaphore_checks=True,
        skip_device_barrier=True,
    ),
)
def _emb_concat_t(tblT_hbm, idx_hbm, chgT_hbm, tail_hbm, dmap_hbm, outT_hbm,
                  din_v, idx_v, orow_v, chg_v, tail_v, slab_v, idx_sh,
                  sem, sem_out, sem_idx, sem_slab2):
    sid = lax.axis_index("s")
    wid = sid * NC + lax.axis_index("c")
    zero16 = lax.iota(jnp.int32, L) * 0
    VH = (VMAIN // 2 // 128) * 128

    def pull_slab(r):
        return (
            pltpu.async_copy(
                tblT_hbm.at[din_v.at[r], pl.ds(0, VH)],
                slab_v.at[:, pl.ds(0, VH)],
                sem,
            ),
            pltpu.async_copy(
                tblT_hbm.at[din_v.at[r], pl.ds(VH, VMAIN - VH)],
                slab_v.at[:, pl.ds(VH, VMAIN - VH)],
                sem_slab2,
            ),
        )

    pltpu.sync_copy(dmap_hbm.at[wid], din_v)
    slab_pull = pull_slab(0)

    @pl.when(sid == 0)
    def _():
        pltpu.sync_copy(idx_hbm, idx_sh)

    pltpu.sync_copy(tail_hbm.at[wid], tail_v)
    idx_pending = pltpu.async_copy(
        idx_hbm.at[pl.ds(0, IDX_CHUNK)], idx_v.at[0], sem_idx
    )
    plsc.subcore_barrier()
    out_pending = None
    for r in range(ROUNDS):
        d = wid + NW * r
        if r > 0:
            slab_pull = pull_slab(r)
            for p in range(2):
                base = (wid * 2 + p) * CHG_COLS
                pltpu.sync_copy(chgT_hbm.at[:, pl.ds(base, CHG_COLS)], chg_v)
                pltpu.sync_copy(chg_v, outT_hbm.at[pl.ds(EMB_DIM, CHG),
                                                   pl.ds(base, CHG_COLS)])
        if out_pending is not None:
            out_pending.wait()
        if r > 0:
            idx_pending = pltpu.async_copy(
                idx_sh.at[pl.ds(0, IDX_CHUNK)], idx_v.at[0], sem_idx
            )
        for c_ in slab_pull:
            c_.wait()
        slab_v[0, pl.ds(VMAIN, L)] = tail_v[r, pl.ds(0, L)]
        slab_v[0, pl.ds(VMAIN + L, L)] = tail_v[r, pl.ds(L, L)]
        for cb in range(NCB):
            idx_pending.wait()
            if cb + 1 < NCB:
                idx_pending = pltpu.async_copy(
                    idx_sh.at[pl.ds((cb + 1) * IDX_CHUNK, IDX_CHUNK)],
                    idx_v.at[(cb + 1) % 2],
                    sem_idx,
                )

            def body(k, cb=cb):
                vidx = idx_v[cb % 2, pl.ds(k, L)]
                vals = plsc.load_gather(slab_v, [zero16, vidx])
                orow_v[0, pl.ds(cb * IDX_CHUNK + k, L)] = vals

            plsc.parallel_loop(0, IDX_CHUNK, step=L, unroll=32)(body)
        out_pending = pltpu.async_copy(orow_v, outT_hbm.at[din_v.at[r]], sem_out)
    out_pending.wait()


def kernel(atom_types, charge, pos, emb_table):
    idx = jnp.reshape(atom_types.astype(jnp.int32), (N,))
    tail = jnp.transpose(
        jnp.reshape(emb_table[VMAIN:, :].T, (ROUNDS, NW, VTAIL)), (1, 0, 2)
    )
    dmap = jnp.reshape(
        jnp.arange(EMB_DIM, dtype=jnp.int32), (ROUNDS, NW)
    ).T.reshape(NW, ROUNDS, 1)
    outT = _emb_concat_t(emb_table.T, idx, charge.T, tail, dmap)
    return outT.T.astype(pos.dtype)


# speedup vs baseline: 1.0630x; 1.0630x over previous
import functools

import jax
import jax.numpy as jnp
from jax import lax
from jax.experimental import pallas as pl
from jax.experimental.pallas import tpu as pltpu
from jax.experimental.pallas import tpu_sc as plsc

N = 16384
VOCAB = 100000
VMAIN = (VOCAB // 128) * 128
VTAIL = VOCAB - VMAIN
EMB_DIM = 64
CHG = 8
OUT_DIM = EMB_DIM + CHG
NC, NS = 2, 16
NW = NC * NS
L = 16
IDX_CHUNK = 2048
NCB = N // IDX_CHUNK
ROUNDS = EMB_DIM // NW
CHG_COLS = N // NW // 2


@functools.partial(
    pl.kernel,
    mesh=plsc.VectorSubcoreMesh(core_axis_name="c", subcore_axis_name="s"),
    out_type=jax.ShapeDtypeStruct((OUT_DIM, N), jnp.float32),
    scratch_types=[
        pltpu.VMEM((ROUNDS, 1), jnp.int32),
        pltpu.VMEM((2, IDX_CHUNK), jnp.int32),
        pltpu.VMEM((1, N), jnp.float32),
        pltpu.VMEM((CHG, CHG_COLS), jnp.float32),
        pltpu.VMEM((ROUNDS, VTAIL), jnp.float32),
        pltpu.VMEM((1, VOCAB), jnp.float32),
        pltpu.VMEM_SHARED((N,), jnp.int32),
        pltpu.SemaphoreType.DMA,
        pltpu.SemaphoreType.DMA,
        pltpu.SemaphoreType.DMA,
        pltpu.SemaphoreType.DMA,
    ],
    compiler_params=pltpu.CompilerParams(
        needs_layout_passes=False,
        disable_bounds_checks=True,
        disable_semaphore_checks=True,
        skip_device_barrier=True,
    ),
)
def _emb_concat_t(tblT_hbm, idx_hbm, chgT_hbm, tail_hbm, dmap_hbm, outT_hbm,
                  din_v, idx_v, orow_v, chg_v, tail_v, slab_v, idx_sh,
                  sem, sem_out, sem_idx, sem_slab2):
    sid = lax.axis_index("s")
    wid = sid * NC + lax.axis_index("c")
    zero16 = lax.iota(jnp.int32, L) * 0
    VH = (VMAIN // 2 // 128) * 128

    def pull_slab(r):
        return (
            pltpu.async_copy(
                tblT_hbm.at[din_v.at[r], pl.ds(0, VH)],
                slab_v.at[:, pl.ds(0, VH)],
                sem,
            ),
            pltpu.async_copy(
                tblT_hbm.at[din_v.at[r], pl.ds(VH, VMAIN - VH)],
                slab_v.at[:, pl.ds(VH, VMAIN - VH)],
                sem_slab2,
            ),
        )

    pltpu.sync_copy(dmap_hbm.at[wid], din_v)
    slab_pull = pull_slab(0)

    @pl.when(sid == 0)
    def _():
        pltpu.sync_copy(idx_hbm, idx_sh)

    pltpu.sync_copy(tail_hbm.at[wid], tail_v)
    idx_pending = pltpu.async_copy(
        idx_hbm.at[pl.ds(0, IDX_CHUNK)], idx_v.at[0], sem_idx
    )
    plsc.subcore_barrier()
    out_pending = None
    for r in range(ROUNDS):
        d = wid + NW * r
        if r > 0:
            slab_pull = pull_slab(r)
            for p in range(2):
                base = (wid * 2 + p) * CHG_COLS
                pltpu.sync_copy(chgT_hbm.at[:, pl.ds(base, CHG_COLS)], chg_v)
                pltpu.sync_copy(chg_v, outT_hbm.at[pl.ds(EMB_DIM, CHG),
                                                   pl.ds(base, CHG_COLS)])
        if out_pending is not None:
            out_pending.wait()
        if r > 0:
            idx_pending = pltpu.async_copy(
                idx_sh.at[pl.ds(0, IDX_CHUNK)], idx_v.at[0], sem_idx
            )
        for c_ in slab_pull:
            c_.wait()
        slab_v[0, pl.ds(VMAIN, L)] = tail_v[r, pl.ds(0, L)]
        slab_v[0, pl.ds(VMAIN + L, L)] = tail_v[r, pl.ds(L, L)]
        for cb in range(NCB):
            idx_pending.wait()
            if cb + 1 < NCB:
                idx_pending = pltpu.async_copy(
                    idx_sh.at[pl.ds((cb + 1) * IDX_CHUNK, IDX_CHUNK)],
                    idx_v.at[(cb + 1) % 2],
                    sem_idx,
                )

            def body(k, cb=cb):
                vidx = idx_v[cb % 2, pl.ds(k, L)]
                vals = plsc.load_gather(slab_v, [zero16, vidx])
                orow_v[0, pl.ds(cb * IDX_CHUNK + k, L)] = vals

            plsc.parallel_loop(0, IDX_CHUNK, step=L, unroll=16)(body)
        out_pending = pltpu.async_copy(orow_v, outT_hbm.at[din_v.at[r]], sem_out)
    out_pending.wait()


def kernel(atom_types, charge, pos, emb_table):
    idx = jnp.reshape(atom_types.astype(jnp.int32), (N,))
    tail = jnp.transpose(
        jnp.reshape(emb_table[VMAIN:, :].T, (ROUNDS, NW, VTAIL)), (1, 0, 2)
    )
    dmap = jnp.reshape(
        jnp.arange(EMB_DIM, dtype=jnp.int32), (ROUNDS, NW)
    ).T.reshape(NW, ROUNDS, 1)
    outT = _emb_concat_t(emb_table.T, idx, charge.T, tail, dmap)
    return outT.T.astype(pos.dtype)
